# baseline (speedup 1.0000x reference)
import functools

import jax
import jax.numpy as jnp
from jax import lax
from jax.experimental import pallas as pl
from jax.experimental.pallas import tpu as pltpu
from jax.experimental.pallas import tpu_sc as plsc


def _conv2d(x, w, stride, pad):
    return lax.conv_general_dilated(x, w, (stride, stride), pad,
                                    dimension_numbers=('NCHW', 'OIHW', 'NCHW'))


def _bnorm(x, p):
    return ((x - p['mean'][None, :, None, None])
            / jnp.sqrt(p['var'][None, :, None, None] + 1e-5)
            * p['scale'][None, :, None, None] + p['bias'][None, :, None, None])


def _resblock(x, p, stride):
    out = jax.nn.relu(_bnorm(_conv2d(x, p['conv1'], stride, ((1, 1), (1, 1))), p['bn1']))
    out = _bnorm(_conv2d(out, p['conv2'], 1, ((1, 1), (1, 1))), p['bn2'])
    if 'down_conv' in p:
        sc = _bnorm(_conv2d(x, p['down_conv'], stride, ((0, 0), (0, 0))), p['down_bn'])
    else:
        sc = x
    return jax.nn.relu(out + sc)


def _backbone(x, params):
    h = jax.nn.relu(_bnorm(_conv2d(x, params['conv1'], 2, ((3, 3), (3, 3))), params['bn1']))
    h = lax.reduce_window(h, -jnp.inf, lax.max, (1, 1, 3, 3), (1, 1, 2, 2),
                          ((0, 0), (0, 0), (1, 1), (1, 1)))
    strides = [1, 2, 2, 2]
    for si, stage in enumerate(params['stages']):
        h = _resblock(h, stage[0], strides[si])
        h = _resblock(h, stage[1], 1)
    return h.mean(axis=(2, 3))


_B = 32
_E = 8
_K = 2
_DOUT = 10


def _erf(x):
    s = jnp.sign(x)
    a = jnp.abs(x)
    t = 1.0 / (1.0 + 0.3275911 * a)
    poly = t * (0.254829592 + t * (-0.284496736 + t * (1.421413741
                + t * (-1.453152027 + t * 1.061405429))))
    return s * (1.0 - poly * jnp.exp(-a * a))


def _gelu_exact(x):
    return 0.5 * x * (1.0 + _erf(x * 0.7071067811865476))


def _head_tc_body(h_ref, fcw_ref, fcb_ref, gw_ref, gb_ref, noise_ref,
                  gauss_ref, w1_ref, b1_ref, w2_ref, b2_ref,
                  gate_ref, eo_ref):
    h = h_ref[...]
    feat = jnp.dot(h, fcw_ref[...], preferred_element_type=jnp.float32) + fcb_ref[...]
    gate = jnp.dot(feat, gw_ref[...], preferred_element_type=jnp.float32) + gb_ref[...]
    gate_ref[...] = gate + noise_ref[...] * gauss_ref[...]
    for e in range(_E):
        a = jnp.dot(feat, w1_ref[e], preferred_element_type=jnp.float32) + b1_ref[e]
        a = _gelu_exact(a)
        eo_ref[e] = jnp.dot(a, w2_ref[e], preferred_element_type=jnp.float32) + b2_ref[e]


def _head_tc(h, fcw, fcb, gw, gb, noise, gauss, w1, b1, w2, b2):
    return pl.pallas_call(
        _head_tc_body,
        out_shape=(
            jax.ShapeDtypeStruct((_B, _E), jnp.float32),
            jax.ShapeDtypeStruct((_E, _B, _DOUT), jnp.float32),
        ),
    )(h, fcw, fcb, gw, gb, noise, gauss, w1, b1, w2, b2)


_NC, _NS, _L = 2, 16, 16


def _route_sc_body(gate_hbm, eo_hbm, out_hbm, gate_v, eo_v, out_v):
    t = lax.axis_index("s") * _NC + lax.axis_index("c")
    pltpu.sync_copy(gate_hbm, gate_v)
    pltpu.sync_copy(eo_hbm, eo_v)
    lane = lax.broadcasted_iota(jnp.int32, (_L,), 0)
    gk = plsc.load_gather(gate_v, [t * _E + lane], mask=lane < _E)
    keys = jnp.where(lane < _E, gk, -jnp.inf)
    skeys, sidx = plsc.sort_key_val(keys, lane, descending=True)
    kmask = lane < _K
    ks = jnp.where(kmask, skeys, -jnp.inf)
    ex = jnp.where(kmask, jnp.exp(ks - jnp.max(ks)), 0.0)
    w = ex / jnp.sum(ex)
    v0 = jnp.sum(jnp.where(lane == 0, sidx, 0))
    v1 = jnp.sum(jnp.where(lane == 1, sidx, 0))
    w0 = jnp.sum(jnp.where(lane == 0, w, 0.0))
    w1 = jnp.sum(jnp.where(lane == 1, w, 0.0))
    omask = lane < _DOUT
    row0 = plsc.load_gather(eo_v, [v0 * (_B * _DOUT) + t * _DOUT + lane], mask=omask)
    row1 = plsc.load_gather(eo_v, [v1 * (_B * _DOUT) + t * _DOUT + lane], mask=omask)
    out_v[...] = jnp.where(omask, w0 * row0 + w1 * row1, 0.0)
    pltpu.sync_copy(out_v, out_hbm.at[t])


@functools.cache
def _make_route_sc():
    return pl.kernel(
        _route_sc_body,
        out_type=jax.ShapeDtypeStruct((_B, _L), jnp.float32),
        mesh=plsc.VectorSubcoreMesh(core_axis_name="c", subcore_axis_name="s"),
        compiler_params=pltpu.CompilerParams(needs_layout_passes=False),
        scratch_types=[
            pltpu.VMEM((_B * _E,), jnp.float32),
            pltpu.VMEM((_E * _B * _DOUT,), jnp.float32),
            pltpu.VMEM((_L,), jnp.float32),
        ],
    )


def kernel(x, gauss, params):
    h = _backbone(x, params)
    ex = params['experts']
    w1 = jnp.stack([e['w1'] for e in ex])
    b1 = jnp.stack([e['b1'] for e in ex])
    w2 = jnp.stack([e['w2'] for e in ex])
    b2 = jnp.stack([e['b2'] for e in ex])
    gate, eo = _head_tc(h, params['fc_w'], params['fc_b'], params['gate_w'],
                        params['gate_b'], params['noise'], gauss, w1, b1, w2, b2)
    out = _make_route_sc()(gate.reshape(_B * _E), eo.reshape(_E * _B * _DOUT))
    return out[:, :_DOUT]


# speedup vs baseline: 1.2919x; 1.2919x over previous
import functools

import jax
import jax.numpy as jnp
from jax import lax
from jax.experimental import pallas as pl
from jax.experimental.pallas import tpu as pltpu
from jax.experimental.pallas import tpu_sc as plsc


def _fold_bn(w, bn):
    s = bn['scale'] / jnp.sqrt(bn['var'] + 1e-5)
    w_hwio = jnp.transpose(w, (2, 3, 1, 0)) * s[None, None, None, :]
    b = bn['bias'] - bn['mean'] * s
    return w_hwio, b


def _conv2d(x, w_hwio, stride, pad, bias):
    out = lax.conv_general_dilated(x, w_hwio, (stride, stride), pad,
                                   dimension_numbers=('NHWC', 'HWIO', 'NHWC'))
    return out + bias[None, None, None, :]


def _resblock(x, p, stride):
    w1, b1 = _fold_bn(p['conv1'], p['bn1'])
    w2, b2 = _fold_bn(p['conv2'], p['bn2'])
    out = jax.nn.relu(_conv2d(x, w1, stride, ((1, 1), (1, 1)), b1))
    out = _conv2d(out, w2, 1, ((1, 1), (1, 1)), b2)
    if 'down_conv' in p:
        wd, bd = _fold_bn(p['down_conv'], p['down_bn'])
        sc = _conv2d(x, wd, stride, ((0, 0), (0, 0)), bd)
    else:
        sc = x
    return jax.nn.relu(out + sc)


def _backbone(x, params):
    xt = jnp.transpose(x, (0, 2, 3, 1))
    w1, b1 = _fold_bn(params['conv1'], params['bn1'])
    h = jax.nn.relu(_conv2d(xt, w1, 2, ((3, 3), (3, 3)), b1))
    h = lax.reduce_window(h, -jnp.inf, lax.max, (1, 3, 3, 1), (1, 2, 2, 1),
                          ((0, 0), (1, 1), (1, 1), (0, 0)))
    strides = [1, 2, 2, 2]
    for si, stage in enumerate(params['stages']):
        h = _resblock(h, stage[0], strides[si])
        h = _resblock(h, stage[1], 1)
    return h.mean(axis=(1, 2))


_B = 32
_E = 8
_K = 2
_DOUT = 10


def _erf(x):
    s = jnp.sign(x)
    a = jnp.abs(x)
    t = 1.0 / (1.0 + 0.3275911 * a)
    poly = t * (0.254829592 + t * (-0.284496736 + t * (1.421413741
                + t * (-1.453152027 + t * 1.061405429))))
    return s * (1.0 - poly * jnp.exp(-a * a))


def _gelu_exact(x):
    return 0.5 * x * (1.0 + _erf(x * 0.7071067811865476))


def _head_tc_body(h_ref, fcw_ref, fcb_ref, gw_ref, gb_ref, noise_ref,
                  gauss_ref, w1_ref, b1_ref, w2_ref, b2_ref,
                  gate_ref, eo_ref):
    h = h_ref[...]
    feat = jnp.dot(h, fcw_ref[...], preferred_element_type=jnp.float32) + fcb_ref[...]
    gate = jnp.dot(feat, gw_ref[...], preferred_element_type=jnp.float32) + gb_ref[...]
    gate_ref[...] = gate + noise_ref[...] * gauss_ref[...]
    for e in range(_E):
        a = jnp.dot(feat, w1_ref[e], preferred_element_type=jnp.float32) + b1_ref[e]
        a = _gelu_exact(a)
        eo_ref[e] = jnp.dot(a, w2_ref[e], preferred_element_type=jnp.float32) + b2_ref[e]


def _head_tc(h, fcw, fcb, gw, gb, noise, gauss, w1, b1, w2, b2):
    return pl.pallas_call(
        _head_tc_body,
        out_shape=(
            jax.ShapeDtypeStruct((_B, _E), jnp.float32),
            jax.ShapeDtypeStruct((_E, _B, _DOUT), jnp.float32),
        ),
    )(h, fcw, fcb, gw, gb, noise, gauss, w1, b1, w2, b2)


_NC, _NS, _L = 2, 16, 16


def _route_sc_body(gate_hbm, eo_hbm, out_hbm, gate_v, eo_v, out_v):
    t = lax.axis_index("s") * _NC + lax.axis_index("c")
    pltpu.sync_copy(gate_hbm, gate_v)
    pltpu.sync_copy(eo_hbm, eo_v)
    lane = lax.broadcasted_iota(jnp.int32, (_L,), 0)
    gk = plsc.load_gather(gate_v, [t * _E + lane], mask=lane < _E)
    keys = jnp.where(lane < _E, gk, -jnp.inf)
    skeys, sidx = plsc.sort_key_val(keys, lane, descending=True)
    kmask = lane < _K
    ks = jnp.where(kmask, skeys, -jnp.inf)
    ex = jnp.where(kmask, jnp.exp(ks - jnp.max(ks)), 0.0)
    w = ex / jnp.sum(ex)
    v0 = jnp.sum(jnp.where(lane == 0, sidx, 0))
    v1 = jnp.sum(jnp.where(lane == 1, sidx, 0))
    w0 = jnp.sum(jnp.where(lane == 0, w, 0.0))
    w1 = jnp.sum(jnp.where(lane == 1, w, 0.0))
    omask = lane < _DOUT
    row0 = plsc.load_gather(eo_v, [v0 * (_B * _DOUT) + t * _DOUT + lane], mask=omask)
    row1 = plsc.load_gather(eo_v, [v1 * (_B * _DOUT) + t * _DOUT + lane], mask=omask)
    out_v[...] = jnp.where(omask, w0 * row0 + w1 * row1, 0.0)
    pltpu.sync_copy(out_v, out_hbm.at[t])


@functools.cache
def _make_route_sc():
    return pl.kernel(
        _route_sc_body,
        out_type=jax.ShapeDtypeStruct((_B, _L), jnp.float32),
        mesh=plsc.VectorSubcoreMesh(core_axis_name="c", subcore_axis_name="s"),
        compiler_params=pltpu.CompilerParams(needs_layout_passes=False),
        scratch_types=[
            pltpu.VMEM((_B * _E,), jnp.float32),
            pltpu.VMEM((_E * _B * _DOUT,), jnp.float32),
            pltpu.VMEM((_L,), jnp.float32),
        ],
    )


def kernel(x, gauss, params):
    h = _backbone(x, params)
    ex = params['experts']
    w1 = jnp.stack([e['w1'] for e in ex])
    b1 = jnp.stack([e['b1'] for e in ex])
    w2 = jnp.stack([e['w2'] for e in ex])
    b2 = jnp.stack([e['b2'] for e in ex])
    gate, eo = _head_tc(h, params['fc_w'], params['fc_b'], params['gate_w'],
                        params['gate_b'], params['noise'], gauss, w1, b1, w2, b2)
    out = _make_route_sc()(gate.reshape(_B * _E), eo.reshape(_E * _B * _DOUT))
    return out[:, :_DOUT]
